# baseline (speedup 1.0000x reference)
import functools

import jax
import jax.numpy as jnp
from jax import lax
from jax.experimental import pallas as pl
from jax.experimental.pallas import tpu as pltpu
from jax.experimental.pallas import tpu_sc as plsc

B = 1024
F = 128
N = 100000
BN = 2048
NB = (N + BN - 1) // BN

IBIG = 2**31 - 1


def _lex_lt(av, ai, bv, bi):
    return (av < bv) | ((av == bv) & (ai < bi))


def _top2_body(supp_ref, key_ref, out_ref, v1s, i1s, v2s, i2s):
    j = pl.program_id(0)
    sims = lax.dot_general(
        supp_ref[...], key_ref[...],
        dimension_numbers=(((0,), (1,)), ((), ())),
        preferred_element_type=jnp.float32)
    rowid = lax.broadcasted_iota(jnp.int32, (BN, B), 0)
    sims = lax.cond(
        j == NB - 1,
        lambda: jnp.where(rowid < N - (NB - 1) * BN, sims, jnp.inf),
        lambda: sims)

    m1 = jnp.min(sims, axis=0, keepdims=True)
    i1 = jnp.min(jnp.where(sims <= m1, rowid, IBIG), axis=0, keepdims=True)
    sims2 = jnp.where(rowid == i1, jnp.inf, sims)
    m2 = jnp.min(sims2, axis=0, keepdims=True)
    i2 = jnp.min(jnp.where(sims2 <= m2, rowid, IBIG), axis=0, keepdims=True)
    off = j * BN
    i1 = i1 + off
    i2 = i2 + off

    @pl.when(j == 0)
    def _init():
        v1s[...] = m1
        i1s[...] = i1
        v2s[...] = m2
        i2s[...] = i2

    @pl.when(j > 0)
    def _merge():
        r1v, r1i = v1s[...], i1s[...]
        r2v, r2i = v2s[...], i2s[...]
        take_r = _lex_lt(r1v, r1i, m1, i1)
        n1v = jnp.where(take_r, r1v, m1)
        n1i = jnp.where(take_r, r1i, i1)
        lv = jnp.where(take_r, m1, r1v)
        li = jnp.where(take_r, i1, r1i)
        wv = jnp.where(take_r, r2v, m2)
        wi = jnp.where(take_r, r2i, i2)
        t2 = _lex_lt(lv, li, wv, wi)
        v1s[...] = n1v
        i1s[...] = n1i
        v2s[...] = jnp.where(t2, lv, wv)
        i2s[...] = jnp.where(t2, li, wi)

    out_ref[...] = i2s[...]


def _neighbor_idx(key, support):
    return pl.pallas_call(
        _top2_body,
        grid=(NB,),
        in_specs=[
            pl.BlockSpec((F, BN), lambda j: (0, j)),
            pl.BlockSpec((B, F), lambda j: (0, 0)),
        ],
        out_specs=pl.BlockSpec((1, B), lambda j: (0, 0)),
        out_shape=jax.ShapeDtypeStruct((1, B), jnp.int32),
        scratch_shapes=[
            pltpu.VMEM((1, B), jnp.float32),
            pltpu.VMEM((1, B), jnp.int32),
            pltpu.VMEM((1, B), jnp.float32),
            pltpu.VMEM((1, B), jnp.int32),
        ],
    )(support, key)


_NC, _NS, _L = 2, 16, 16
_NW = _NC * _NS
_BPW = B // _NW


def _sc_gather_dot(support_flat, idx, anchor_t):
    mesh = plsc.VectorSubcoreMesh(core_axis_name="c", subcore_axis_name="s")

    nchunk = F * _BPW // 128
    fpc = 128 // _BPW

    @functools.partial(
        pl.kernel,
        mesh=mesh,
        out_type=jax.ShapeDtypeStruct((B,), jnp.float32),
        scratch_types=[
            pltpu.VMEM((_BPW,), jnp.int32),
            pltpu.VMEM((nchunk, 128), jnp.int32),
            pltpu.VMEM((nchunk, 128), jnp.float32),
            pltpu.VMEM((F, _BPW), jnp.float32),
            pltpu.VMEM((_BPW,), jnp.float32),
            pltpu.SemaphoreType.DMA,
        ],
    )
    def k(supp_hbm, idx_hbm, anc_hbm, out_hbm, idx_v, fidx_v, gath_v, anc_v,
          out_v, sem):
        wid = lax.axis_index("s") * _NC + lax.axis_index("c")
        base = wid * _BPW
        pltpu.sync_copy(idx_hbm.at[pl.ds(base, _BPW)], idx_v)
        pltpu.sync_copy(anc_hbm.at[wid], anc_v)
        ia = idx_v[pl.ds(0, _L)]
        ib = idx_v[pl.ds(_L, _L)]

        def fill(c, _):
            for q in range(fpc):
                off = (fpc * c + q) * N
                fidx_v[c, pl.ds(q * _BPW, _L)] = ia + off
                fidx_v[c, pl.ds(q * _BPW + _L, _L)] = ib + off
            return 0

        lax.fori_loop(0, nchunk, fill, 0)

        def fire(c, _):
            pltpu.async_copy(supp_hbm.at[fidx_v.at[c]], gath_v.at[c], sem)
            return 0

        lax.fori_loop(0, nchunk, fire, 0)

        def drain(c, _):
            pltpu.make_async_copy(
                supp_hbm.at[fidx_v.at[c]], gath_v.at[c], sem).wait()
            return 0

        lax.fori_loop(0, nchunk, drain, 0)

        def acc(c, carry):
            a0, a1 = carry
            for q in range(fpc):
                f = fpc * c + q
                a0 = a0 + anc_v[f, pl.ds(0, _L)] * gath_v[c, pl.ds(q * _BPW, _L)]
                a1 = a1 + anc_v[f, pl.ds(_L, _L)] * gath_v[c, pl.ds(q * _BPW + _L, _L)]
            return (a0, a1)

        zero = jnp.zeros((_L,), jnp.float32)
        a0, a1 = lax.fori_loop(0, nchunk, acc, (zero, zero))
        out_v[pl.ds(0, _L)] = a0
        out_v[pl.ds(_L, _L)] = a1
        pltpu.sync_copy(out_v, out_hbm.at[pl.ds(base, _BPW)])

    return k(support_flat, idx, anchor_t)


def kernel(anchor, key, support):
    idx = _neighbor_idx(key, support).reshape(B)
    support_flat = support.reshape(F * N)
    anchor_t = anchor.T.reshape(F, _NW, _BPW).transpose(1, 0, 2)
    out = _sc_gather_dot(support_flat, idx, anchor_t)
    return out.reshape(B, 1)


# speedup vs baseline: 1.1706x; 1.1706x over previous
import functools

import jax
import jax.numpy as jnp
from jax import lax
from jax.experimental import pallas as pl
from jax.experimental.pallas import tpu as pltpu
from jax.experimental.pallas import tpu_sc as plsc

B = 1024
F = 128
N = 100000
BN = 2048
NBM = 98304 // BN
NT = N - NBM * BN

IBIG = 2**31 - 1


def _lex_lt(av, ai, bv, bi):
    return (av < bv) | ((av == bv) & (ai < bi))


def _block_top2(sims, bn):
    rowid = lax.broadcasted_iota(jnp.int32, (bn, B), 0)
    m1 = jnp.min(sims, axis=0, keepdims=True)
    i1 = jnp.min(jnp.where(sims <= m1, rowid, IBIG), axis=0, keepdims=True)
    sims2 = jnp.where(rowid == i1, jnp.inf, sims)
    m2 = jnp.min(sims2, axis=0, keepdims=True)
    i2 = jnp.min(jnp.where(sims2 <= m2, rowid, IBIG), axis=0, keepdims=True)
    return m1, i1, m2, i2


def _merge_top2(r1v, r1i, r2v, r2i, c1v, c1i, c2v, c2i):
    take_r = _lex_lt(r1v, r1i, c1v, c1i)
    n1v = jnp.where(take_r, r1v, c1v)
    n1i = jnp.where(take_r, r1i, c1i)
    lv = jnp.where(take_r, c1v, r1v)
    li = jnp.where(take_r, c1i, r1i)
    wv = jnp.where(take_r, r2v, c2v)
    wi = jnp.where(take_r, r2i, c2i)
    t2 = _lex_lt(lv, li, wv, wi)
    return n1v, n1i, jnp.where(t2, lv, wv), jnp.where(t2, li, wi)


def _sims_block(supp_ref, key_ref):
    return lax.dot_general(
        supp_ref[...], key_ref[...],
        dimension_numbers=(((0,), (1,)), ((), ())),
        preferred_element_type=jnp.float32)


def _main_body(supp_ref, key_ref, v1o, i1o, v2o, i2o):
    j = pl.program_id(0)
    m1, i1, m2, i2 = _block_top2(_sims_block(supp_ref, key_ref), BN)
    off = j * BN
    i1 = i1 + off
    i2 = i2 + off

    @pl.when(j == 0)
    def _init():
        v1o[...] = m1
        i1o[...] = i1
        v2o[...] = m2
        i2o[...] = i2

    @pl.when(j > 0)
    def _merge():
        n1v, n1i, n2v, n2i = _merge_top2(
            v1o[...], i1o[...], v2o[...], i2o[...], m1, i1, m2, i2)
        v1o[...] = n1v
        i1o[...] = n1i
        v2o[...] = n2v
        i2o[...] = n2i


def _tail_body(supp_ref, key_ref, v1_ref, i1_ref, v2_ref, i2_ref, out_ref):
    m1, i1, m2, i2 = _block_top2(_sims_block(supp_ref, key_ref), NT)
    off = NBM * BN
    _, _, _, n2i = _merge_top2(
        v1_ref[...], i1_ref[...], v2_ref[...], i2_ref[...],
        m1, i1 + off, m2, i2 + off)
    out_ref[...] = n2i


def _neighbor_idx(key, support):
    st = jax.ShapeDtypeStruct((1, B), jnp.int32)
    sv = jax.ShapeDtypeStruct((1, B), jnp.float32)
    v1, i1, v2, i2 = pl.pallas_call(
        _main_body,
        grid=(NBM,),
        in_specs=[
            pl.BlockSpec((F, BN), lambda j: (0, j)),
            pl.BlockSpec((B, F), lambda j: (0, 0)),
        ],
        out_specs=[pl.BlockSpec((1, B), lambda j: (0, 0))] * 4,
        out_shape=[sv, st, sv, st],
    )(support[:, :NBM * BN], key)
    return pl.pallas_call(
        _tail_body,
        in_specs=[
            pl.BlockSpec((F, NT), lambda: (0, 0)),
            pl.BlockSpec((B, F), lambda: (0, 0)),
        ] + [pl.BlockSpec((1, B), lambda: (0, 0))] * 4,
        out_specs=pl.BlockSpec((1, B), lambda: (0, 0)),
        out_shape=st,
    )(support[:, NBM * BN:], key, v1, i1, v2, i2)


_NC, _NS, _L = 2, 16, 16
_NW = _NC * _NS
_BPW = B // _NW


def _sc_gather_dot(support_flat, idx, anchor_t):
    mesh = plsc.VectorSubcoreMesh(core_axis_name="c", subcore_axis_name="s")

    nchunk = F * _BPW // 128
    fpc = 128 // _BPW

    @functools.partial(
        pl.kernel,
        mesh=mesh,
        out_type=jax.ShapeDtypeStruct((B,), jnp.float32),
        scratch_types=[
            pltpu.VMEM((_BPW,), jnp.int32),
            pltpu.VMEM((nchunk, 128), jnp.int32),
            pltpu.VMEM((nchunk, 128), jnp.float32),
            pltpu.VMEM((F, _BPW), jnp.float32),
            pltpu.VMEM((_BPW,), jnp.float32),
            pltpu.SemaphoreType.DMA,
        ],
    )
    def k(supp_hbm, idx_hbm, anc_hbm, out_hbm, idx_v, fidx_v, gath_v, anc_v,
          out_v, sem):
        wid = lax.axis_index("s") * _NC + lax.axis_index("c")
        base = wid * _BPW
        pltpu.sync_copy(idx_hbm.at[pl.ds(base, _BPW)], idx_v)
        pltpu.sync_copy(anc_hbm.at[wid], anc_v)
        ia = idx_v[pl.ds(0, _L)]
        ib = idx_v[pl.ds(_L, _L)]

        def fill(c, _):
            for q in range(fpc):
                off = (fpc * c + q) * N
                fidx_v[c, pl.ds(q * _BPW, _L)] = ia + off
                fidx_v[c, pl.ds(q * _BPW + _L, _L)] = ib + off
            return 0

        lax.fori_loop(0, nchunk, fill, 0)

        def fire(c, _):
            pltpu.async_copy(supp_hbm.at[fidx_v.at[c]], gath_v.at[c], sem)
            return 0

        lax.fori_loop(0, nchunk, fire, 0)

        def drain(c, _):
            pltpu.make_async_copy(
                supp_hbm.at[fidx_v.at[c]], gath_v.at[c], sem).wait()
            return 0

        lax.fori_loop(0, nchunk, drain, 0)

        def acc(c, carry):
            a0, a1 = carry
            for q in range(fpc):
                f = fpc * c + q
                a0 = a0 + anc_v[f, pl.ds(0, _L)] * gath_v[c, pl.ds(q * _BPW, _L)]
                a1 = a1 + anc_v[f, pl.ds(_L, _L)] * gath_v[c, pl.ds(q * _BPW + _L, _L)]
            return (a0, a1)

        zero = jnp.zeros((_L,), jnp.float32)
        a0, a1 = lax.fori_loop(0, nchunk, acc, (zero, zero))
        out_v[pl.ds(0, _L)] = a0
        out_v[pl.ds(_L, _L)] = a1
        pltpu.sync_copy(out_v, out_hbm.at[pl.ds(base, _BPW)])

    return k(support_flat, idx, anchor_t)


def kernel(anchor, key, support):
    idx = _neighbor_idx(key, support).reshape(B)
    support_flat = support.reshape(F * N)
    anchor_t = anchor.T.reshape(F, _NW, _BPW).transpose(1, 0, 2)
    out = _sc_gather_dot(support_flat, idx, anchor_t)
    return out.reshape(B, 1)


# speedup vs baseline: 1.4214x; 1.2142x over previous
import functools

import jax
import jax.numpy as jnp
from jax import lax
from jax.experimental import pallas as pl
from jax.experimental.pallas import tpu as pltpu
from jax.experimental.pallas import tpu_sc as plsc

B = 1024
F = 128
N = 100000
BN = 2048
NBM = 98304 // BN
NT = N - NBM * BN

IBIG = 2**31 - 1


CH = 32


def _lex_lt(av, ai, bv, bi):
    return (av < bv) | ((av == bv) & (ai < bi))


def _merge_top2(r1v, r1i, r2v, r2i, c1v, c1i, c2v, c2i):
    take_r = _lex_lt(r1v, r1i, c1v, c1i)
    n1v = jnp.where(take_r, r1v, c1v)
    n1i = jnp.where(take_r, r1i, c1i)
    lv = jnp.where(take_r, c1v, r1v)
    li = jnp.where(take_r, c1i, r1i)
    wv = jnp.where(take_r, r2v, c2v)
    wi = jnp.where(take_r, r2i, c2i)
    t2 = _lex_lt(lv, li, wv, wi)
    return n1v, n1i, jnp.where(t2, lv, wv), jnp.where(t2, li, wi)


def _sims_block(supp_ref, key_ref):
    return lax.dot_general(
        supp_ref[...], key_ref[...],
        dimension_numbers=(((0,), (1,)), ((), ())),
        preferred_element_type=jnp.float32)


def _scan_rows(sims_ref, nrows, rbase, riota, carry):
    def chunk(c, carry):
        v1, i1, v2, i2 = carry
        xs = sims_ref[pl.ds(c * CH, CH), :]
        for u in range(CH // 8):
            x = lax.slice_in_dim(xs, u * 8, (u + 1) * 8)
            r = riota + (rbase + c * CH + u * 8)
            lt1 = x < v1
            c2 = x < v2
            v2n = jnp.where(c2, x, v2)
            i2n = jnp.where(c2, r, i2)
            v2 = jnp.where(lt1, v1, v2n)
            i2 = jnp.where(lt1, i1, i2n)
            v1 = jnp.where(lt1, x, v1)
            i1 = jnp.where(lt1, r, i1)
        return (v1, i1, v2, i2)

    return lax.fori_loop(0, nrows // CH, chunk, carry)


def _main_body(supp_ref, key_ref, v1o, i1o, v2o, i2o, sims_scr):
    j = pl.program_id(0)
    sims_scr[...] = _sims_block(supp_ref, key_ref)

    @pl.when(j == 0)
    def _init():
        v1o[...] = jnp.full((8, B), jnp.inf, jnp.float32)
        i1o[...] = jnp.full((8, B), IBIG, jnp.int32)
        v2o[...] = jnp.full((8, B), jnp.inf, jnp.float32)
        i2o[...] = jnp.full((8, B), IBIG, jnp.int32)

    riota = lax.broadcasted_iota(jnp.int32, (8, B), 0)
    carry = (v1o[...], i1o[...], v2o[...], i2o[...])
    v1, i1, v2, i2 = _scan_rows(sims_scr, BN, j * BN, riota, carry)
    v1o[...] = v1
    i1o[...] = i1
    v2o[...] = v2
    i2o[...] = i2


def _tail_body(supp_ref, key_ref, v1_ref, i1_ref, v2_ref, i2_ref, out_ref,
               sims_scr):
    sims_scr[...] = _sims_block(supp_ref, key_ref)
    riota = lax.broadcasted_iota(jnp.int32, (8, B), 0)
    carry = (v1_ref[...], i1_ref[...], v2_ref[...], i2_ref[...])
    v1, i1, v2, i2 = _scan_rows(sims_scr, NT, NBM * BN, riota, carry)
    for h in (4, 2, 1):
        v1, i1, v2, i2 = _merge_top2(
            v1[:h], i1[:h], v2[:h], i2[:h],
            v1[h:2 * h], i1[h:2 * h], v2[h:2 * h], i2[h:2 * h])
    out_ref[...] = i2


def _neighbor_idx(key, support):
    st = jax.ShapeDtypeStruct((8, B), jnp.int32)
    sv = jax.ShapeDtypeStruct((8, B), jnp.float32)
    state_spec = pl.BlockSpec((8, B), lambda *_: (0, 0))
    v1, i1, v2, i2 = pl.pallas_call(
        _main_body,
        grid=(NBM,),
        in_specs=[
            pl.BlockSpec((F, BN), lambda j: (0, j)),
            pl.BlockSpec((B, F), lambda j: (0, 0)),
        ],
        out_specs=[state_spec] * 4,
        out_shape=[sv, st, sv, st],
        scratch_shapes=[pltpu.VMEM((BN, B), jnp.float32)],
    )(support, key)
    return pl.pallas_call(
        _tail_body,
        grid=(1,),
        in_specs=[
            pl.BlockSpec((F, BN), lambda j: (0, NBM)),
            pl.BlockSpec((B, F), lambda j: (0, 0)),
        ] + [state_spec] * 4,
        out_specs=pl.BlockSpec((1, B), lambda j: (0, 0)),
        out_shape=jax.ShapeDtypeStruct((1, B), jnp.int32),
        scratch_shapes=[pltpu.VMEM((BN, B), jnp.float32)],
    )(support, key, v1, i1, v2, i2)


_NC, _NS, _L = 2, 16, 16
_NW = _NC * _NS
_BPW = B // _NW


def _sc_gather_dot(support_flat, idx, anchor_t):
    mesh = plsc.VectorSubcoreMesh(core_axis_name="c", subcore_axis_name="s")

    nchunk = F * _BPW // 128
    fpc = 128 // _BPW

    @functools.partial(
        pl.kernel,
        mesh=mesh,
        out_type=jax.ShapeDtypeStruct((B,), jnp.float32),
        scratch_types=[
            pltpu.VMEM((_BPW,), jnp.int32),
            pltpu.VMEM((nchunk, 128), jnp.int32),
            pltpu.VMEM((nchunk, 128), jnp.float32),
            pltpu.VMEM((F, _BPW), jnp.float32),
            pltpu.VMEM((_BPW,), jnp.float32),
            pltpu.SemaphoreType.DMA,
        ],
    )
    def k(supp_hbm, idx_hbm, anc_hbm, out_hbm, idx_v, fidx_v, gath_v, anc_v,
          out_v, sem):
        wid = lax.axis_index("s") * _NC + lax.axis_index("c")
        base = wid * _BPW
        pltpu.sync_copy(idx_hbm.at[pl.ds(base, _BPW)], idx_v)
        pltpu.sync_copy(anc_hbm.at[wid], anc_v)
        ia = idx_v[pl.ds(0, _L)]
        ib = idx_v[pl.ds(_L, _L)]

        def fill(c, _):
            for q in range(fpc):
                off = (fpc * c + q) * N
                fidx_v[c, pl.ds(q * _BPW, _L)] = ia + off
                fidx_v[c, pl.ds(q * _BPW + _L, _L)] = ib + off
            return 0

        lax.fori_loop(0, nchunk, fill, 0)

        def fire(c, _):
            pltpu.async_copy(supp_hbm.at[fidx_v.at[c]], gath_v.at[c], sem)
            return 0

        lax.fori_loop(0, nchunk, fire, 0)

        def drain(c, _):
            pltpu.make_async_copy(
                supp_hbm.at[fidx_v.at[c]], gath_v.at[c], sem).wait()
            return 0

        lax.fori_loop(0, nchunk, drain, 0)

        def acc(c, carry):
            a0, a1 = carry
            for q in range(fpc):
                f = fpc * c + q
                a0 = a0 + anc_v[f, pl.ds(0, _L)] * gath_v[c, pl.ds(q * _BPW, _L)]
                a1 = a1 + anc_v[f, pl.ds(_L, _L)] * gath_v[c, pl.ds(q * _BPW + _L, _L)]
            return (a0, a1)

        zero = jnp.zeros((_L,), jnp.float32)
        a0, a1 = lax.fori_loop(0, nchunk, acc, (zero, zero))
        out_v[pl.ds(0, _L)] = a0
        out_v[pl.ds(_L, _L)] = a1
        pltpu.sync_copy(out_v, out_hbm.at[pl.ds(base, _BPW)])

    return k(support_flat, idx, anchor_t)


def kernel(anchor, key, support):
    idx = _neighbor_idx(key, support).reshape(B)
    support_flat = support.reshape(F * N)
    anchor_t = anchor.T.reshape(F, _NW, _BPW).transpose(1, 0, 2)
    out = _sc_gather_dot(support_flat, idx, anchor_t)
    return out.reshape(B, 1)
